# Initial kernel scaffold; baseline (speedup 1.0000x reference)
import jax
import jax.numpy as jnp
from jax.experimental import pallas as pl


def kernel(h, codebook):
    raise NotImplementedError("write your pallas kernel here")


# speedup vs baseline: 1.0021x; 1.0021x over previous
import functools

import jax
import jax.numpy as jnp
from jax import lax
from jax.experimental import pallas as pl

_BLK = 512
_K = 1024
_D = 256


def _vq_block(flat_ref, cb_ref, q_ref, idx_ref):
    x = flat_ref[...]
    cb = cb_ref[...]
    scores = lax.dot_general(
        x, cb, (((1,), (1,)), ((), ())),
        preferred_element_type=jnp.float32)
    c2 = jnp.sum(cb * cb, axis=1)
    x2 = jnp.sum(x * x, axis=1, keepdims=True)
    d = (x2 - 2.0 * scores) + c2[None, :]
    dmin = jnp.min(d, axis=1, keepdims=True)
    iota = lax.broadcasted_iota(jnp.int32, (_BLK, _K), 1)
    idx = jnp.min(jnp.where(d == dmin, iota, _K), axis=1).astype(jnp.int32)
    onehot = (lax.broadcasted_iota(jnp.int32, (_BLK, _K), 1)
              == idx[:, None]).astype(jnp.float32)
    q = lax.dot_general(
        onehot, cb, (((1,), (0,)), ((), ())),
        precision=lax.Precision.HIGHEST,
        preferred_element_type=jnp.float32)
    q_ref[...] = x + (q - x)
    idx_ref[...] = idx[None, None, :]


@functools.partial(jax.jit, static_argnames=())
def kernel(h, codebook):
    b, t, d = h.shape
    bt = b * t
    flat = h.reshape(bt, d)
    nblk = bt // _BLK
    q, idx = pl.pallas_call(
        _vq_block,
        grid=(nblk,),
        in_specs=[
            pl.BlockSpec((_BLK, d), lambda i: (i, 0)),
            pl.BlockSpec((_K, d), lambda i: (0, 0)),
        ],
        out_specs=[
            pl.BlockSpec((_BLK, d), lambda i: (i, 0)),
            pl.BlockSpec((1, 1, _BLK), lambda i: (i, 0, 0)),
        ],
        out_shape=[
            jax.ShapeDtypeStruct((bt, d), jnp.float32),
            jax.ShapeDtypeStruct((nblk, 1, _BLK), jnp.int32),
        ],
    )(flat, codebook)
    return q.reshape(b, t, d), idx.reshape(b, t)


# speedup vs baseline: 1.5021x; 1.4990x over previous
import functools

import jax
import jax.numpy as jnp
from jax import lax
from jax.experimental import pallas as pl
from jax.experimental.pallas import tpu as pltpu
from jax.experimental.pallas import tpu_sc as plsc

_BLK = 512
_K = 1024
_D = 256
_BT = 32768

_NC = 2
_NS = 16
_NW = _NC * _NS
_B_PER_W = _BT // _NW
_CHUNK = 128
_NCHUNK = _B_PER_W // _CHUNK


def _argmin_block(flat_ref, cb_ref, idx_ref):
    x = flat_ref[...]
    cb = cb_ref[...]
    scores = lax.dot_general(
        x, cb, (((1,), (1,)), ((), ())),
        preferred_element_type=jnp.float32)
    c2 = jnp.sum(cb * cb, axis=1)
    x2 = jnp.sum(x * x, axis=1, keepdims=True)
    d = (x2 - 2.0 * scores) + c2[None, :]
    dmin = jnp.min(d, axis=1, keepdims=True)
    iota = lax.broadcasted_iota(jnp.int32, (_BLK, _K), 1)
    idx = jnp.min(jnp.where(d == dmin, iota, _K), axis=1).astype(jnp.int32)
    idx_ref[...] = idx[None, None, :]


def _compute_indices(flat, codebook):
    nblk = _BT // _BLK
    idx3 = pl.pallas_call(
        _argmin_block,
        grid=(nblk,),
        in_specs=[
            pl.BlockSpec((_BLK, _D), lambda i: (i, 0)),
            pl.BlockSpec((_K, _D), lambda i: (0, 0)),
        ],
        out_specs=pl.BlockSpec((1, 1, _BLK), lambda i: (i, 0, 0)),
        out_shape=jax.ShapeDtypeStruct((nblk, 1, _BLK), jnp.int32),
    )(flat, codebook)
    return idx3.reshape(_BT)


@functools.partial(
    pl.kernel,
    mesh=plsc.VectorSubcoreMesh(core_axis_name="c", subcore_axis_name="s"),
    out_type=jax.ShapeDtypeStruct((_BT, _D), jnp.float32),
    scratch_types=[
        pltpu.VMEM((_CHUNK,), jnp.int32),
        pltpu.VMEM((_CHUNK, _D), jnp.float32),
        pltpu.SemaphoreType.DMA,
    ],
)
def _sc_gather(table_hbm, idx_hbm, out_hbm, idx_v, rows_v, sem):
    wid = lax.axis_index("s") * _NC + lax.axis_index("c")
    base = wid * _B_PER_W
    for j in range(_NCHUNK):
        off = base + j * _CHUNK
        pltpu.sync_copy(idx_hbm.at[pl.ds(off, _CHUNK)], idx_v)
        pltpu.async_copy(table_hbm.at[idx_v], rows_v, sem).wait()
        pltpu.sync_copy(rows_v, out_hbm.at[pl.ds(off, _CHUNK)])


@jax.jit
def kernel(h, codebook):
    b, t, d = h.shape
    flat = h.reshape(_BT, d)
    idx = _compute_indices(flat, codebook)
    q = _sc_gather(codebook, idx)
    return q.reshape(b, t, d), idx.reshape(b, t)


# speedup vs baseline: 1.7308x; 1.1523x over previous
import functools

import jax
import jax.numpy as jnp
from jax import lax
from jax.experimental import pallas as pl
from jax.experimental.pallas import tpu as pltpu
from jax.experimental.pallas import tpu_sc as plsc

_BLK = 512
_K = 1024
_D = 256
_BT = 32768

_NC = 2
_NS = 16
_NW = _NC * _NS
_B_PER_W = _BT // _NW
_CHUNK = 128
_NCHUNK = _B_PER_W // _CHUNK


def _argmin_block(flat_ref, cb_ref, idx_ref, c2_ref):
    x = flat_ref[...]
    cb = cb_ref[...]

    @pl.when(pl.program_id(0) == 0)
    def _():
        c2_ref[...] = jnp.sum(cb * cb, axis=1)[None, :]

    s2 = lax.dot_general(
        x * -2.0, cb, (((1,), (1,)), ((), ())),
        preferred_element_type=jnp.float32)
    x2 = jnp.sum(x * x, axis=1, keepdims=True)
    d = (x2 + s2) + c2_ref[...]
    dmin = jnp.min(d, axis=1, keepdims=True)
    iota_f = lax.broadcasted_iota(
        jnp.int32, (_BLK, _K), 1).astype(jnp.float32)
    idx = jnp.min(jnp.where(d == dmin, iota_f, float(_K)),
                  axis=1, keepdims=True)
    idx_ref[...] = idx.astype(jnp.int32)


def _compute_indices(flat, codebook):
    nblk = _BT // _BLK
    idx_col = pl.pallas_call(
        _argmin_block,
        grid=(nblk,),
        in_specs=[
            pl.BlockSpec((_BLK, _D), lambda i: (i, 0)),
            pl.BlockSpec((_K, _D), lambda i: (0, 0)),
        ],
        out_specs=pl.BlockSpec((_BLK, 1), lambda i: (i, 0)),
        out_shape=jax.ShapeDtypeStruct((_BT, 1), jnp.int32),
        scratch_shapes=[pltpu.VMEM((1, _K), jnp.float32)],
    )(flat, codebook)
    return idx_col.reshape(_BT)


@functools.partial(
    pl.kernel,
    mesh=plsc.VectorSubcoreMesh(core_axis_name="c", subcore_axis_name="s"),
    out_type=jax.ShapeDtypeStruct((_BT, _D), jnp.float32),
    scratch_types=[
        pltpu.VMEM((_CHUNK,), jnp.int32),
        pltpu.VMEM((_CHUNK, _D), jnp.float32),
        pltpu.SemaphoreType.DMA,
    ],
)
def _sc_gather(table_hbm, idx_hbm, out_hbm, idx_v, rows_v, sem):
    wid = lax.axis_index("s") * _NC + lax.axis_index("c")
    base = wid * _B_PER_W
    for j in range(_NCHUNK):
        off = base + j * _CHUNK
        pltpu.sync_copy(idx_hbm.at[pl.ds(off, _CHUNK)], idx_v)
        pltpu.async_copy(table_hbm.at[idx_v], rows_v, sem).wait()
        pltpu.sync_copy(rows_v, out_hbm.at[pl.ds(off, _CHUNK)])


@jax.jit
def kernel(h, codebook):
    b, t, d = h.shape
    flat = h.reshape(_BT, d)
    idx = _compute_indices(flat, codebook)
    q = _sc_gather(codebook, idx)
    return q.reshape(b, t, d), idx.reshape(b, t)


# speedup vs baseline: 1.7696x; 1.0224x over previous
import functools

import jax
import jax.numpy as jnp
from jax import lax
from jax.experimental import pallas as pl
from jax.experimental.pallas import tpu as pltpu
from jax.experimental.pallas import tpu_sc as plsc

_BLK = 512
_K = 1024
_D = 256
_BT = 32768

_NC = 2
_NS = 16
_NW = _NC * _NS
_B_PER_W = _BT // _NW
_CHUNK = 128
_NCHUNK = _B_PER_W // _CHUNK


def _argmin_block(flat_ref, cb_ref, idx_ref):
    x = flat_ref[...]
    cb = cb_ref[...]
    c2 = jnp.sum(cb * cb, axis=1)
    s2 = lax.dot_general(
        x * -2.0, cb, (((1,), (1,)), ((), ())),
        preferred_element_type=jnp.float32)
    x2 = jnp.sum(x * x, axis=1, keepdims=True)
    d = (x2 + s2) + c2[None, :]
    dmin = jnp.min(d, axis=1, keepdims=True)
    iota_f = lax.broadcasted_iota(
        jnp.int32, (_BLK, _K), 1).astype(jnp.float32)
    idx = jnp.min(jnp.where(d == dmin, iota_f, float(_K)),
                  axis=1, keepdims=True)
    idx_ref[...] = idx.astype(jnp.int32)


def _compute_indices(flat, codebook):
    nblk = _BT // _BLK
    idx_col = pl.pallas_call(
        _argmin_block,
        grid=(nblk,),
        in_specs=[
            pl.BlockSpec((_BLK, _D), lambda i: (i, 0)),
            pl.BlockSpec((_K, _D), lambda i: (0, 0)),
        ],
        out_specs=pl.BlockSpec((_BLK, 1), lambda i: (i, 0)),
        out_shape=jax.ShapeDtypeStruct((_BT, 1), jnp.int32),
        compiler_params=pltpu.CompilerParams(
            dimension_semantics=("parallel",)),
    )(flat, codebook)
    return idx_col.reshape(_BT)


@functools.partial(
    pl.kernel,
    mesh=plsc.VectorSubcoreMesh(core_axis_name="c", subcore_axis_name="s"),
    out_type=jax.ShapeDtypeStruct((_BT, _D), jnp.float32),
    scratch_types=[
        pltpu.VMEM((_CHUNK,), jnp.int32),
        pltpu.VMEM((_CHUNK, _D), jnp.float32),
        pltpu.SemaphoreType.DMA,
    ],
)
def _sc_gather(table_hbm, idx_hbm, out_hbm, idx_v, rows_v, sem):
    wid = lax.axis_index("s") * _NC + lax.axis_index("c")
    base = wid * _B_PER_W
    for j in range(_NCHUNK):
        off = base + j * _CHUNK
        pltpu.sync_copy(idx_hbm.at[pl.ds(off, _CHUNK)], idx_v)
        pltpu.async_copy(table_hbm.at[idx_v], rows_v, sem).wait()
        pltpu.sync_copy(rows_v, out_hbm.at[pl.ds(off, _CHUNK)])


@jax.jit
def kernel(h, codebook):
    b, t, d = h.shape
    flat = h.reshape(_BT, d)
    idx = _compute_indices(flat, codebook)
    q = _sc_gather(codebook, idx)
    return q.reshape(b, t, d), idx.reshape(b, t)


# speedup vs baseline: 1.7858x; 1.0092x over previous
import functools

import jax
import jax.numpy as jnp
from jax import lax
from jax.experimental import pallas as pl
from jax.experimental.pallas import tpu as pltpu
from jax.experimental.pallas import tpu_sc as plsc

_BLK = 512
_K = 1024
_D = 256
_BT = 32768
_KT = 128

_NC = 2
_NS = 16
_NW = _NC * _NS
_B_PER_W = _BT // _NW
_CHUNK = 128
_NCHUNK = _B_PER_W // _CHUNK


def _c2_once(cb_ref, c2_ref):
    cb = cb_ref[...]
    c2_ref[...] = jnp.sum(cb * cb, axis=1)[None, :]


def _argmin_block(flat_ref, cb_ref, c2_ref, idx_ref):
    x = flat_ref[...]
    cb = cb_ref[...]
    c2 = c2_ref[...]
    s2 = lax.dot_general(
        x * -2.0, cb, (((1,), (1,)), ((), ())),
        preferred_element_type=jnp.float32)
    x2 = jnp.sum(x * x, axis=1, keepdims=True)

    lane = lax.broadcasted_iota(
        jnp.int32, (_BLK, _KT), 1).astype(jnp.float32)
    m = None
    im = None
    for t in range(_K // _KT):
        d_t = ((x2 + s2[:, t * _KT:(t + 1) * _KT])
               + c2[:, t * _KT:(t + 1) * _KT])
        if t == 0:
            m = d_t
            im = lane
        else:
            better = d_t < m
            im = jnp.where(better, lane + float(t * _KT), im)
            m = jnp.minimum(d_t, m)
    dmin = jnp.min(m, axis=1, keepdims=True)
    idx = jnp.min(jnp.where(m == dmin, im, float(_K)),
                  axis=1, keepdims=True)
    idx_ref[...] = idx.astype(jnp.int32)


def _compute_indices(flat, codebook):
    c2 = pl.pallas_call(
        _c2_once,
        out_shape=jax.ShapeDtypeStruct((1, _K), jnp.float32),
    )(codebook)
    nblk = _BT // _BLK
    idx_col = pl.pallas_call(
        _argmin_block,
        grid=(nblk,),
        in_specs=[
            pl.BlockSpec((_BLK, _D), lambda i: (i, 0)),
            pl.BlockSpec((_K, _D), lambda i: (0, 0)),
            pl.BlockSpec((1, _K), lambda i: (0, 0)),
        ],
        out_specs=pl.BlockSpec((_BLK, 1), lambda i: (i, 0)),
        out_shape=jax.ShapeDtypeStruct((_BT, 1), jnp.int32),
        compiler_params=pltpu.CompilerParams(
            dimension_semantics=("parallel",)),
    )(flat, codebook, c2)
    return idx_col.reshape(_BT)


@functools.partial(
    pl.kernel,
    mesh=plsc.VectorSubcoreMesh(core_axis_name="c", subcore_axis_name="s"),
    out_type=jax.ShapeDtypeStruct((_BT, _D), jnp.float32),
    scratch_types=[
        pltpu.VMEM((_CHUNK,), jnp.int32),
        pltpu.VMEM((_CHUNK, _D), jnp.float32),
        pltpu.SemaphoreType.DMA,
    ],
)
def _sc_gather(table_hbm, idx_hbm, out_hbm, idx_v, rows_v, sem):
    wid = lax.axis_index("s") * _NC + lax.axis_index("c")
    base = wid * _B_PER_W
    for j in range(_NCHUNK):
        off = base + j * _CHUNK
        pltpu.sync_copy(idx_hbm.at[pl.ds(off, _CHUNK)], idx_v)
        pltpu.async_copy(table_hbm.at[idx_v], rows_v, sem).wait()
        pltpu.sync_copy(rows_v, out_hbm.at[pl.ds(off, _CHUNK)])


@jax.jit
def kernel(h, codebook):
    b, t, d = h.shape
    flat = h.reshape(_BT, d)
    idx = _compute_indices(flat, codebook)
    q = _sc_gather(codebook, idx)
    return q.reshape(b, t, d), idx.reshape(b, t)
